# baseline (device time: 47168 ns/iter reference)
import jax
import jax.numpy as jnp
from jax import lax
from jax.experimental import pallas as pl
from jax.experimental.pallas import tpu as pltpu

N_DEV = 4
SEG = 8


def kernel(x):
    _, m, n_tot = x.shape
    n_out = n_tot // N_DEV
    half = n_out // 2
    mseg = m // SEG

    def body(x_ref, out_ref, comm_r, comm_l, contrib_r, contrib_l,
             send_r, recv_r, send_l, recv_l):
        my = lax.axis_index("i")
        left = (my - 1) % N_DEV
        right = (my + 1) % N_DEV

        barrier_sem = pltpu.get_barrier_semaphore()
        for nbr in (left, right):
            pl.semaphore_signal(
                barrier_sem, inc=1,
                device_id=(nbr,), device_id_type=pl.DeviceIdType.MESH,
            )
        pl.semaphore_wait(barrier_sem, 2)

        def half0(c, rows):
            return x_ref[0, rows, pl.ds(c * n_out, half)].astype(jnp.bfloat16)

        def half1(c, rows):
            return x_ref[0, rows, pl.ds(c * n_out + half, half)].astype(
                jnp.bfloat16)

        def rdma(direction, h, s):
            comm, ssem, rsem, dst_dev = (
                (comm_r, send_r, recv_r, right) if direction == 0
                else (comm_l, send_l, recv_l, left)
            )
            src_slot = 3 if h == 0 else h - 1
            rows = pl.ds(s * mseg, mseg)
            return pltpu.make_async_remote_copy(
                src_ref=comm.at[src_slot, rows, :],
                dst_ref=comm.at[h, rows, :],
                send_sem=ssem.at[h, s],
                recv_sem=rsem.at[h, s],
                device_id=(dst_dev,),
                device_id_type=pl.DeviceIdType.MESH,
            )

        full = pl.ds(0, m)

        for s in range(SEG):
            rows = pl.ds(s * mseg, mseg)
            comm_r[3, rows, :] = half0(left, rows)
            rdma(0, 0, s).start()
            comm_l[3, rows, :] = half1(right, rows)
            rdma(1, 0, s).start()

        contrib_r[0, :, :] = half0((my - 2) % N_DEV, full)
        contrib_l[0, :, :] = half1((my + 2) % N_DEV, full)
        contrib_r[1, :, :] = half0((my - 3) % N_DEV, full)
        contrib_l[1, :, :] = half1((my + 3) % N_DEV, full)

        for h in range(N_DEV - 2):
            for s in range(SEG):
                rows = pl.ds(s * mseg, mseg)
                rdma(0, h, s).wait_recv()
                comm_r[h, rows, :] = (
                    comm_r[h, rows, :] + contrib_r[h, rows, :]
                )
                rdma(0, h + 1, s).start()
                rdma(1, h, s).wait_recv()
                comm_l[h, rows, :] = (
                    comm_l[h, rows, :] + contrib_l[h, rows, :]
                )
                rdma(1, h + 1, s).start()

        hl = N_DEV - 2
        for s in range(SEG):
            rows = pl.ds(s * mseg, mseg)
            rdma(0, hl, s).wait_recv()
            out_ref[rows, :half] = (
                comm_r[hl, rows, :].astype(jnp.float32)
                + x_ref[0, rows, pl.ds(my * n_out, half)]
            )
            rdma(1, hl, s).wait_recv()
            out_ref[rows, half:] = (
                comm_l[hl, rows, :].astype(jnp.float32)
                + x_ref[0, rows, pl.ds(my * n_out + half, half)]
            )

        for h in range(N_DEV - 1):
            for s in range(SEG):
                rdma(0, h, s).wait_send()
                rdma(1, h, s).wait_send()

    return pl.pallas_call(
        body,
        out_shape=jax.ShapeDtypeStruct((m, n_out), jnp.float32),
        in_specs=[pl.BlockSpec(memory_space=pltpu.VMEM)],
        out_specs=pl.BlockSpec(memory_space=pltpu.VMEM),
        scratch_shapes=[
            pltpu.VMEM((4, m, half), jnp.bfloat16),
            pltpu.VMEM((4, m, half), jnp.bfloat16),
            pltpu.VMEM((2, m, half), jnp.bfloat16),
            pltpu.VMEM((2, m, half), jnp.bfloat16),
            pltpu.SemaphoreType.DMA((3, SEG)),
            pltpu.SemaphoreType.DMA((3, SEG)),
            pltpu.SemaphoreType.DMA((3, SEG)),
            pltpu.SemaphoreType.DMA((3, SEG)),
        ],
        compiler_params=pltpu.CompilerParams(collective_id=0),
    )(x)


# device time: 46805 ns/iter; 1.0078x vs baseline; 1.0078x over previous
import jax
import jax.numpy as jnp
from jax import lax
from jax.experimental import pallas as pl
from jax.experimental.pallas import tpu as pltpu

N_DEV = 4
SEG = 4


def kernel(x):
    _, m, n_tot = x.shape
    n_out = n_tot // N_DEV
    half = n_out // 2
    mseg = m // SEG

    def body(x_ref, out_ref, comm_r, comm_l, contrib_r, contrib_l,
             send_r, recv_r, send_l, recv_l):
        my = lax.axis_index("i")

        def run(k):
            left = (k - 1) % N_DEV
            right = (k + 1) % N_DEV

            barrier_sem = pltpu.get_barrier_semaphore()
            for nbr in (left, right):
                pl.semaphore_signal(
                    barrier_sem, inc=1,
                    device_id=(nbr,), device_id_type=pl.DeviceIdType.MESH,
                )
            pl.semaphore_wait(barrier_sem, 2)

            def half0(c, rows):
                return x_ref[0, rows, c * n_out:c * n_out + half].astype(
                    jnp.bfloat16)

            def half1(c, rows):
                return x_ref[
                    0, rows, c * n_out + half:(c + 1) * n_out
                ].astype(jnp.bfloat16)

            def rdma(direction, h, s):
                comm, ssem, rsem, dst_dev = (
                    (comm_r, send_r, recv_r, right) if direction == 0
                    else (comm_l, send_l, recv_l, left)
                )
                src_slot = 3 if h == 0 else h - 1
                rows = pl.ds(s * mseg, mseg)
                return pltpu.make_async_remote_copy(
                    src_ref=comm.at[src_slot, rows, :],
                    dst_ref=comm.at[h, rows, :],
                    send_sem=ssem.at[h, s],
                    recv_sem=rsem.at[h, s],
                    device_id=(dst_dev,),
                    device_id_type=pl.DeviceIdType.MESH,
                )

            full = slice(None)

            for s in range(SEG):
                rows = pl.ds(s * mseg, mseg)
                comm_r[3, rows, :] = half0(left, rows)
                rdma(0, 0, s).start()
                comm_l[3, rows, :] = half1(right, rows)
                rdma(1, 0, s).start()

            contrib_r[0, :, :] = half0((k - 2) % N_DEV, full)
            contrib_l[0, :, :] = half1((k + 2) % N_DEV, full)
            contrib_r[1, :, :] = half0((k - 3) % N_DEV, full)
            contrib_l[1, :, :] = half1((k + 3) % N_DEV, full)

            for h in range(N_DEV - 2):
                for s in range(SEG):
                    rows = pl.ds(s * mseg, mseg)
                    rdma(0, h, s).wait_recv()
                    comm_r[h, rows, :] = (
                        comm_r[h, rows, :] + contrib_r[h, rows, :]
                    )
                    rdma(0, h + 1, s).start()
                    rdma(1, h, s).wait_recv()
                    comm_l[h, rows, :] = (
                        comm_l[h, rows, :] + contrib_l[h, rows, :]
                    )
                    rdma(1, h + 1, s).start()

            hl = N_DEV - 2
            for s in range(SEG):
                rows = pl.ds(s * mseg, mseg)
                rdma(0, hl, s).wait_recv()
                out_ref[rows, :half] = (
                    comm_r[hl, rows, :].astype(jnp.float32)
                    + x_ref[0, rows, k * n_out:k * n_out + half]
                )
                rdma(1, hl, s).wait_recv()
                out_ref[rows, half:] = (
                    comm_l[hl, rows, :].astype(jnp.float32)
                    + x_ref[0, rows, k * n_out + half:(k + 1) * n_out]
                )

            for h in range(N_DEV - 1):
                for s in range(SEG):
                    rdma(0, h, s).wait_send()
                    rdma(1, h, s).wait_send()

        for k in range(N_DEV):
            pl.when(my == k)(lambda k=k: run(k))

    return pl.pallas_call(
        body,
        out_shape=jax.ShapeDtypeStruct((m, n_out), jnp.float32),
        in_specs=[pl.BlockSpec(memory_space=pltpu.VMEM)],
        out_specs=pl.BlockSpec(memory_space=pltpu.VMEM),
        scratch_shapes=[
            pltpu.VMEM((4, m, half), jnp.bfloat16),
            pltpu.VMEM((4, m, half), jnp.bfloat16),
            pltpu.VMEM((2, m, half), jnp.bfloat16),
            pltpu.VMEM((2, m, half), jnp.bfloat16),
            pltpu.SemaphoreType.DMA((3, SEG)),
            pltpu.SemaphoreType.DMA((3, SEG)),
            pltpu.SemaphoreType.DMA((3, SEG)),
            pltpu.SemaphoreType.DMA((3, SEG)),
        ],
        compiler_params=pltpu.CompilerParams(collective_id=0),
    )(x)
